# baseline (speedup 1.0000x reference)
import functools

import jax
import jax.numpy as jnp
from jax import lax
from jax.experimental import pallas as pl
from jax.experimental.pallas import tpu as pltpu
from jax.experimental.pallas import tpu_sc as plsc

N_TOKEN = 8192
N_EMBED = 256
N_CODE = 8192

TBLK = 256
CBLK = 512


def _argmin_body(x_ref, wt_ref, xs_ref, ws_ref, idx_ref, best_ref, bidx_ref):
    j = pl.program_id(1)
    x = x_ref[...]
    wt = wt_ref[...]
    mm = lax.dot_general(
        x.astype(jnp.bfloat16), wt.astype(jnp.bfloat16),
        (((1,), (0,)), ((), ())),
        preferred_element_type=jnp.float32)
    d = (xs_ref[...] + ws_ref[...]) - 2.0 * mm
    bmin = jnp.min(d, axis=1, keepdims=True)
    col = lax.broadcasted_iota(jnp.int32, (TBLK, CBLK), 1)
    barg = jnp.min(jnp.where(d == bmin, col, N_CODE),
                   axis=1, keepdims=True)
    gidx = barg + j * CBLK

    @pl.when(j == 0)
    def _init():
        best_ref[...] = bmin
        bidx_ref[...] = gidx

    @pl.when(j > 0)
    def _update():
        prev = best_ref[...]
        upd = bmin < prev
        best_ref[...] = jnp.where(upd, bmin, prev)
        bidx_ref[...] = jnp.where(upd, gidx, bidx_ref[...])

    @pl.when(j == pl.num_programs(1) - 1)
    def _emit():
        idx_ref[...] = bidx_ref[...]


def _argmin_indices(x, WT, xs, ws):
    grid = (N_TOKEN // TBLK, N_CODE // CBLK)
    return pl.pallas_call(
        _argmin_body,
        grid=grid,
        in_specs=[
            pl.BlockSpec((TBLK, N_EMBED), lambda i, j: (i, 0)),
            pl.BlockSpec((N_EMBED, CBLK), lambda i, j: (0, j)),
            pl.BlockSpec((TBLK, 1), lambda i, j: (i, 0)),
            pl.BlockSpec((1, CBLK), lambda i, j: (0, j)),
        ],
        out_specs=pl.BlockSpec((TBLK, 1), lambda i, j: (i, 0)),
        out_shape=jax.ShapeDtypeStruct((N_TOKEN, 1), jnp.int32),
        scratch_shapes=[
            pltpu.VMEM((TBLK, 1), jnp.float32),
            pltpu.VMEM((TBLK, 1), jnp.int32),
        ],
        compiler_params=pltpu.CompilerParams(
            dimension_semantics=("parallel", "arbitrary")),
    )(x, WT, xs, ws)


def _make_gather():
    info = plsc.get_sparse_core_info()
    nc, ns = info.num_cores, info.num_subcores
    nw = nc * ns
    b_per_w = N_TOKEN // nw
    mesh = plsc.VectorSubcoreMesh(core_axis_name="c", subcore_axis_name="s")

    @functools.partial(
        pl.kernel, mesh=mesh,
        out_type=jax.ShapeDtypeStruct((N_TOKEN, N_EMBED), jnp.float32),
        scratch_types=[
            pltpu.VMEM((b_per_w,), jnp.int32),
            pltpu.VMEM((b_per_w, N_EMBED), jnp.float32),
            pltpu.SemaphoreType.DMA,
        ],
    )
    def gather(table_hbm, idx_hbm, out_hbm, idx_v, rows_v, sem):
        wid = lax.axis_index("s") * nc + lax.axis_index("c")
        base = wid * b_per_w
        pltpu.sync_copy(idx_hbm.at[pl.ds(base, b_per_w)], idx_v)
        pltpu.async_copy(table_hbm.at[idx_v], rows_v, sem).wait()
        pltpu.sync_copy(rows_v, out_hbm.at[pl.ds(base, b_per_w)])

    return gather


def kernel(inputs, W):
    size = inputs.shape
    x = inputs.reshape(-1, N_EMBED)
    xs = (x ** 2).sum(axis=-1, keepdims=True)
    ws = (W ** 2).sum(axis=-1)[None, :]
    indices = _argmin_indices(x, W.T, xs, ws).reshape(-1)
    quantized = _make_gather()(W, indices)
    return quantized.reshape(size)


# speedup vs baseline: 2.1377x; 2.1377x over previous
import functools

import jax
import jax.numpy as jnp
from jax import lax
from jax.experimental import pallas as pl
from jax.experimental.pallas import tpu as pltpu
from jax.experimental.pallas import tpu_sc as plsc

N_TOKEN = 8192
N_EMBED = 256
N_CODE = 8192

TBLK = 256
CBLK = 2048


def _argmin_body(x_ref, wt_ref, xs_ref, ws_ref, idx_ref, best_ref, bidx_ref):
    j = pl.program_id(1)
    x = x_ref[...]
    wt = wt_ref[...]
    mm = lax.dot_general(
        x.astype(jnp.bfloat16), wt.astype(jnp.bfloat16),
        (((1,), (0,)), ((), ())),
        preferred_element_type=jnp.float32)
    d = (xs_ref[...] + ws_ref[...]) - 2.0 * mm
    bmin = jnp.min(d, axis=1, keepdims=True)
    col = lax.broadcasted_iota(jnp.int32, (TBLK, CBLK), 1)
    barg = jnp.min(jnp.where(d == bmin, col, N_CODE),
                   axis=1, keepdims=True)
    gidx = barg + j * CBLK

    @pl.when(j == 0)
    def _init():
        best_ref[...] = bmin
        bidx_ref[...] = gidx

    @pl.when(j > 0)
    def _update():
        prev = best_ref[...]
        upd = bmin < prev
        best_ref[...] = jnp.where(upd, bmin, prev)
        bidx_ref[...] = jnp.where(upd, gidx, bidx_ref[...])

    @pl.when(j == pl.num_programs(1) - 1)
    def _emit():
        idx_ref[...] = bidx_ref[...]


def _argmin_indices(x, WT, xs, ws):
    grid = (N_TOKEN // TBLK, N_CODE // CBLK)
    return pl.pallas_call(
        _argmin_body,
        grid=grid,
        in_specs=[
            pl.BlockSpec((TBLK, N_EMBED), lambda i, j: (i, 0)),
            pl.BlockSpec((N_EMBED, CBLK), lambda i, j: (0, j)),
            pl.BlockSpec((TBLK, 1), lambda i, j: (i, 0)),
            pl.BlockSpec((1, CBLK), lambda i, j: (0, j)),
        ],
        out_specs=pl.BlockSpec((TBLK, 1), lambda i, j: (i, 0)),
        out_shape=jax.ShapeDtypeStruct((N_TOKEN, 1), jnp.int32),
        scratch_shapes=[
            pltpu.VMEM((TBLK, 1), jnp.float32),
            pltpu.VMEM((TBLK, 1), jnp.int32),
        ],
        compiler_params=pltpu.CompilerParams(
            dimension_semantics=("parallel", "arbitrary")),
    )(x, WT, xs, ws)


def _make_gather():
    info = plsc.get_sparse_core_info()
    nc, ns = info.num_cores, info.num_subcores
    nw = nc * ns
    b_per_w = N_TOKEN // nw
    mesh = plsc.VectorSubcoreMesh(core_axis_name="c", subcore_axis_name="s")

    @functools.partial(
        pl.kernel, mesh=mesh,
        out_type=jax.ShapeDtypeStruct((N_TOKEN, N_EMBED), jnp.float32),
        scratch_types=[
            pltpu.VMEM((b_per_w,), jnp.int32),
            pltpu.VMEM((b_per_w, N_EMBED), jnp.float32),
            pltpu.SemaphoreType.DMA,
        ],
    )
    def gather(table_hbm, idx_hbm, out_hbm, idx_v, rows_v, sem):
        wid = lax.axis_index("s") * nc + lax.axis_index("c")
        base = wid * b_per_w
        pltpu.sync_copy(idx_hbm.at[pl.ds(base, b_per_w)], idx_v)
        pltpu.async_copy(table_hbm.at[idx_v], rows_v, sem).wait()
        pltpu.sync_copy(rows_v, out_hbm.at[pl.ds(base, b_per_w)])

    return gather


def kernel(inputs, W):
    size = inputs.shape
    x = inputs.reshape(-1, N_EMBED)
    xs = (x ** 2).sum(axis=-1, keepdims=True)
    ws = (W ** 2).sum(axis=-1)[None, :]
    indices = _argmin_indices(x, W.T, xs, ws).reshape(-1)
    quantized = _make_gather()(W, indices)
    return quantized.reshape(size)


# speedup vs baseline: 2.9377x; 1.3742x over previous
import functools

import jax
import jax.numpy as jnp
from jax import lax
from jax.experimental import pallas as pl
from jax.experimental.pallas import tpu as pltpu
from jax.experimental.pallas import tpu_sc as plsc

N_TOKEN = 8192
N_EMBED = 256
N_CODE = 8192

TBLK = 512
CBLK = 2048


def _argmin_body(x_ref, wt_ref, xs_ref, ws_ref, idx_ref, best_ref, bidx_ref):
    j = pl.program_id(1)
    x = x_ref[...]
    wt = wt_ref[...]
    mm = lax.dot_general(
        x.astype(jnp.bfloat16), wt.astype(jnp.bfloat16),
        (((1,), (0,)), ((), ())),
        preferred_element_type=jnp.float32)
    d = (xs_ref[...] + ws_ref[...]) - 2.0 * mm
    bmin = jnp.min(d, axis=1, keepdims=True)
    col = lax.broadcasted_iota(jnp.int32, (TBLK, CBLK), 1)
    barg = jnp.min(jnp.where(d == bmin, col, N_CODE),
                   axis=1, keepdims=True)
    gidx = barg + j * CBLK

    @pl.when(j == 0)
    def _init():
        best_ref[...] = bmin
        bidx_ref[...] = gidx

    @pl.when(j > 0)
    def _update():
        prev = best_ref[...]
        upd = bmin < prev
        best_ref[...] = jnp.where(upd, bmin, prev)
        bidx_ref[...] = jnp.where(upd, gidx, bidx_ref[...])

    @pl.when(j == pl.num_programs(1) - 1)
    def _emit():
        idx_ref[...] = bidx_ref[...]


def _argmin_indices(x, WT, xs, ws):
    grid = (N_TOKEN // TBLK, N_CODE // CBLK)
    return pl.pallas_call(
        _argmin_body,
        grid=grid,
        in_specs=[
            pl.BlockSpec((TBLK, N_EMBED), lambda i, j: (i, 0)),
            pl.BlockSpec((N_EMBED, CBLK), lambda i, j: (0, j)),
            pl.BlockSpec((TBLK, 1), lambda i, j: (i, 0)),
            pl.BlockSpec((1, CBLK), lambda i, j: (0, j)),
        ],
        out_specs=pl.BlockSpec((TBLK, 1), lambda i, j: (i, 0)),
        out_shape=jax.ShapeDtypeStruct((N_TOKEN, 1), jnp.int32),
        scratch_shapes=[
            pltpu.VMEM((TBLK, 1), jnp.float32),
            pltpu.VMEM((TBLK, 1), jnp.int32),
        ],
        compiler_params=pltpu.CompilerParams(
            dimension_semantics=("parallel", "arbitrary")),
    )(x, WT, xs, ws)


def _make_gather():
    info = plsc.get_sparse_core_info()
    nc, ns = info.num_cores, info.num_subcores
    nw = nc * ns
    b_per_w = N_TOKEN // nw
    mesh = plsc.VectorSubcoreMesh(core_axis_name="c", subcore_axis_name="s")

    @functools.partial(
        pl.kernel, mesh=mesh,
        out_type=jax.ShapeDtypeStruct((N_TOKEN, N_EMBED), jnp.float32),
        scratch_types=[
            pltpu.VMEM((b_per_w,), jnp.int32),
            pltpu.VMEM((b_per_w, N_EMBED), jnp.float32),
            pltpu.SemaphoreType.DMA,
        ],
    )
    def gather(table_hbm, idx_hbm, out_hbm, idx_v, rows_v, sem):
        wid = lax.axis_index("s") * nc + lax.axis_index("c")
        base = wid * b_per_w
        pltpu.sync_copy(idx_hbm.at[pl.ds(base, b_per_w)], idx_v)
        pltpu.async_copy(table_hbm.at[idx_v], rows_v, sem).wait()
        pltpu.sync_copy(rows_v, out_hbm.at[pl.ds(base, b_per_w)])

    return gather


def kernel(inputs, W):
    size = inputs.shape
    x = inputs.reshape(-1, N_EMBED)
    xs = (x ** 2).sum(axis=-1, keepdims=True)
    ws = (W ** 2).sum(axis=-1)[None, :]
    indices = _argmin_indices(x, W.T, xs, ws).reshape(-1)
    quantized = _make_gather()(W, indices)
    return quantized.reshape(size)
